# baseline (speedup 1.0000x reference)
import functools

import jax
import jax.numpy as jnp
from jax import lax
from jax.experimental import pallas as pl
from jax.experimental.pallas import tpu as pltpu
from jax.experimental.pallas import tpu_sc as plsc

COL_A = 17
COL_B = 503
BLK_A = 0
BLK_B = 384
WIN_A = 16
WIN_B = 112
LANE_A = COL_A - BLK_A - WIN_A
LANE_B = COL_B - BLK_B - WIN_B

N_ROWS = 16384
N_COLS = 1024

NUM_CORES = 2
NUM_SUBCORES = 16
NUM_WORKERS = NUM_CORES * NUM_SUBCORES
ROWS_PER_WORKER = N_ROWS // NUM_WORKERS
CHUNK = 16
NUM_CHUNKS = ROWS_PER_WORKER // CHUNK
RING = 4
LAG = 2
HALF = ROWS_PER_WORKER // 2
HALF_CHUNKS = NUM_CHUNKS // 2

_mesh = plsc.VectorSubcoreMesh(
    core_axis_name="c",
    subcore_axis_name="s",
    num_cores=NUM_CORES,
    num_subcores=NUM_SUBCORES,
)


@functools.partial(
    pl.kernel,
    out_type=jax.ShapeDtypeStruct((N_ROWS, N_COLS), jnp.float32),
    mesh=_mesh,
    scratch_types=(
        [pltpu.VMEM_SHARED((NUM_SUBCORES, RING, CHUNK // 8, 8, N_COLS), jnp.float32)]
        + [pltpu.VMEM((HALF, 128), jnp.float32) for _ in range(2)]
        + [pltpu.SemaphoreType.DMA for _ in range(2 * RING + 2)]
    ),
    compiler_params=pltpu.CompilerParams(
        use_tc_tiling_on_sc=True, needs_layout_passes=False
    ),
)
def _swap_columns(x_hbm, out_hbm, spmem, blk_a, blk_b, *sems):
    sem_in = sems[:RING]
    sem_out = sems[RING : 2 * RING]
    sem_blk_in, sem_blk_out = sems[2 * RING :]

    x_flat = x_hbm.reshape(N_ROWS // 8, 8, N_COLS)
    out_flat = out_hbm.reshape(N_ROWS // 8, 8, N_COLS)

    cid = lax.axis_index("c")
    sid = lax.axis_index("s")
    wid = sid * NUM_CORES + cid
    r0 = wid * ROWS_PER_WORKER

    def stage_half(h):
        rows_h = pl.ds(r0 + h * HALF, HALF)
        ca = pltpu.async_copy(x_hbm.at[rows_h, pl.ds(BLK_A, 128)], blk_a, sem_blk_in)
        cb = pltpu.async_copy(x_hbm.at[rows_h, pl.ds(BLK_B, 128)], blk_b, sem_blk_in)
        return ca, cb

    def fix_half():
        lane = lax.iota(jnp.int32, 16)
        bcast_a = jnp.full((16,), LANE_A, jnp.int32)
        bcast_b = jnp.full((16,), LANE_B, jnp.int32)

        def body(t, carry):
            for j in range(8):
                r = t * 8 + j
                va = blk_a[r, pl.ds(WIN_A, 16)]
                vb = blk_b[r, pl.ds(WIN_B, 16)]
                a_at_swap = va.at[bcast_a].get(mode="promise_in_bounds")
                b_at_swap = vb.at[bcast_b].get(mode="promise_in_bounds")
                blk_a[r, pl.ds(WIN_A, 16)] = jnp.where(lane == LANE_A, b_at_swap, va)
                blk_b[r, pl.ds(WIN_B, 16)] = jnp.where(lane == LANE_B, a_at_swap, vb)
            return carry

        lax.fori_loop(0, HALF // 8, body, 0)

    def write_half(h):
        rows_h = pl.ds(r0 + h * HALF, HALF)
        wa = pltpu.async_copy(blk_a, out_hbm.at[rows_h, pl.ds(BLK_A, 128)], sem_blk_out)
        wb = pltpu.async_copy(blk_b, out_hbm.at[rows_h, pl.ds(BLK_B, 128)], sem_blk_out)
        return wa, wb

    def chunk_flat(c):
        return pl.ds((r0 + c * CHUNK) // 8, CHUNK // 8)

    ins = [None] * RING
    outs = [None] * RING

    def ring_issue(c):
        b = c % RING
        if outs[b] is not None:
            outs[b].wait()
            outs[b] = None
        ins[b] = pltpu.async_copy(
            x_flat.at[chunk_flat(c)], spmem.at[sid, b], sem_in[b]
        )

    def ring_drain(j):
        bj = j % RING
        ins[bj].wait()
        outs[bj] = pltpu.async_copy(
            spmem.at[sid, bj], out_flat.at[chunk_flat(j)], sem_out[bj]
        )

    ca, cb = stage_half(0)
    for c in range(LAG):
        ring_issue(c)
    ca.wait()
    cb.wait()
    fix_half()
    for c in range(LAG, HALF_CHUNKS):
        ring_issue(c)
        ring_drain(c - LAG)
    for j in range(HALF_CHUNKS - LAG, HALF_CHUNKS):
        ring_drain(j)
    for b in range(RING):
        if outs[b] is not None:
            outs[b].wait()
            outs[b] = None

    for c in range(HALF_CHUNKS, HALF_CHUNKS + LAG):
        ring_issue(c)
    wa, wb = write_half(0)
    wa.wait()
    wb.wait()
    ca, cb = stage_half(1)
    ca.wait()
    cb.wait()
    fix_half()
    for c in range(HALF_CHUNKS + LAG, NUM_CHUNKS):
        ring_issue(c)
        ring_drain(c - LAG)
    for j in range(NUM_CHUNKS - LAG, NUM_CHUNKS):
        ring_drain(j)
    for b in range(RING):
        if outs[b] is not None:
            outs[b].wait()
            outs[b] = None

    wa, wb = write_half(1)
    wa.wait()
    wb.wait()


def kernel(X):
    return _swap_columns(X)


# speedup vs baseline: 1.0061x; 1.0061x over previous
import functools

import jax
import jax.numpy as jnp
from jax import lax
from jax.experimental import pallas as pl
from jax.experimental.pallas import tpu as pltpu
from jax.experimental.pallas import tpu_sc as plsc

COL_A = 17
COL_B = 503
BLK_A = 0
BLK_B = 384
WIN_A = 16
WIN_B = 112
LANE_A = COL_A - BLK_A - WIN_A
LANE_B = COL_B - BLK_B - WIN_B

N_ROWS = 16384
N_COLS = 1024

NUM_CORES = 2
NUM_SUBCORES = 16
NUM_WORKERS = NUM_CORES * NUM_SUBCORES
ROWS_PER_WORKER = N_ROWS // NUM_WORKERS
CHUNK = 16
NUM_CHUNKS = ROWS_PER_WORKER // CHUNK
RING = 4
LAG = 2
HALF = ROWS_PER_WORKER // 2
HALF_CHUNKS = NUM_CHUNKS // 2

_mesh = plsc.VectorSubcoreMesh(
    core_axis_name="c",
    subcore_axis_name="s",
    num_cores=NUM_CORES,
    num_subcores=NUM_SUBCORES,
)


@functools.partial(
    pl.kernel,
    out_type=jax.ShapeDtypeStruct((N_ROWS, N_COLS), jnp.float32),
    mesh=_mesh,
    scratch_types=(
        [pltpu.VMEM_SHARED((NUM_SUBCORES, RING, CHUNK // 8, 8, N_COLS), jnp.float32)]
        + [pltpu.VMEM((HALF, 128), jnp.float32) for _ in range(2)]
        + [pltpu.SemaphoreType.DMA for _ in range(2 * RING + 2)]
    ),
    compiler_params=pltpu.CompilerParams(
        use_tc_tiling_on_sc=True, needs_layout_passes=False
    ),
)
def _swap_columns(x_hbm, out_hbm, spmem, blk_a, blk_b, *sems):
    sem_in = sems[:RING]
    sem_out = sems[RING : 2 * RING]
    sem_blk_in, sem_blk_out = sems[2 * RING :]

    x_flat = x_hbm.reshape(N_ROWS // 8, 8, N_COLS)
    out_flat = out_hbm.reshape(N_ROWS // 8, 8, N_COLS)

    cid = lax.axis_index("c")
    sid = lax.axis_index("s")
    wid = sid * NUM_CORES + cid
    r0 = wid * ROWS_PER_WORKER

    def stage_half(h):
        rows_h = pl.ds(r0 + h * HALF, HALF)
        ca = pltpu.async_copy(x_hbm.at[rows_h, pl.ds(BLK_A, 128)], blk_a, sem_blk_in)
        cb = pltpu.async_copy(x_hbm.at[rows_h, pl.ds(BLK_B, 128)], blk_b, sem_blk_in)
        return ca, cb

    def fix_half():
        lane = lax.iota(jnp.int32, 16)
        bcast_a = jnp.full((16,), LANE_A, jnp.int32)
        bcast_b = jnp.full((16,), LANE_B, jnp.int32)

        def body(t, carry):
            for j in range(8):
                r = t * 8 + j
                va = blk_a[r, pl.ds(WIN_A, 16)]
                vb = blk_b[r, pl.ds(WIN_B, 16)]
                a_at_swap = va.at[bcast_a].get(mode="promise_in_bounds")
                b_at_swap = vb.at[bcast_b].get(mode="promise_in_bounds")
                blk_a[r, pl.ds(WIN_A, 16)] = jnp.where(lane == LANE_A, b_at_swap, va)
                blk_b[r, pl.ds(WIN_B, 16)] = jnp.where(lane == LANE_B, a_at_swap, vb)
            return carry

        lax.fori_loop(0, HALF // 8, body, 0)

    def write_half(h):
        rows_h = pl.ds(r0 + h * HALF, HALF)
        wa = pltpu.async_copy(blk_a, out_hbm.at[rows_h, pl.ds(BLK_A, 128)], sem_blk_out)
        wb = pltpu.async_copy(blk_b, out_hbm.at[rows_h, pl.ds(BLK_B, 128)], sem_blk_out)
        return wa, wb

    def chunk_flat(c):
        return pl.ds((r0 + c * CHUNK) // 8, CHUNK // 8)

    ins = [None] * RING
    outs = [None] * RING
    sp_pend = []
    sp_count = [0]

    def issue(c):
        b = sp_count[0] % RING
        sp_count[0] += 1
        if outs[b] is not None:
            outs[b].wait()
            outs[b] = None
        ins[b] = pltpu.async_copy(
            x_flat.at[chunk_flat(c)], spmem.at[sid, b], sem_in[b]
        )
        sp_pend.append((c, b))

    def drain_ready():
        while len(sp_pend) > LAG:
            j, bj = sp_pend.pop(0)
            ins[bj].wait()
            outs[bj] = pltpu.async_copy(
                spmem.at[sid, bj], out_flat.at[chunk_flat(j)], sem_out[bj]
            )

    def flush():
        while sp_pend:
            j, bj = sp_pend.pop(0)
            ins[bj].wait()
            outs[bj] = pltpu.async_copy(
                spmem.at[sid, bj], out_flat.at[chunk_flat(j)], sem_out[bj]
            )
        for b in range(RING):
            if outs[b] is not None:
                outs[b].wait()
                outs[b] = None

    ca, cb = stage_half(0)
    for c in range(LAG):
        issue(c)
    ca.wait()
    cb.wait()
    fix_half()
    for c in range(LAG, HALF_CHUNKS):
        issue(c)
        drain_ready()
    flush()

    for c in range(HALF_CHUNKS, HALF_CHUNKS + LAG):
        issue(c)
    wa, wb = write_half(0)
    wa.wait()
    wb.wait()
    ca, cb = stage_half(1)
    ca.wait()
    cb.wait()
    fix_half()
    for c in range(HALF_CHUNKS + LAG, NUM_CHUNKS):
        issue(c)
        drain_ready()
    flush()

    wa, wb = write_half(1)
    wa.wait()
    wb.wait()


def kernel(X):
    return _swap_columns(X)
